# baseline (speedup 1.0000x reference)
import functools

import jax
import jax.numpy as jnp
from jax import lax
from jax.experimental import pallas as pl
from jax.experimental.pallas import tpu as pltpu
from jax.experimental.pallas import tpu_sc as plsc

N = 10000
E = 320000
G = 8
DN = 128
DE = 16
EO = 16
NO = 128

GROUP = 128
NGROUPS = E // GROUP
NCORES = 2
NSUB = 16
NW = NCORES * NSUB

NPAD = 10240
NPASS = 4
SEG = NPAD // NPASS
PACK = SEG // 8
CHUNK = 400
EPT = E // NW
NCHUNK = EPT // CHUNK
NGRP = CHUNK // 16

TE = 2000
TN = 2000


def _ln(x):
    m = jnp.mean(x, axis=-1, keepdims=True)
    v = jnp.mean((x - m) ** 2, axis=-1, keepdims=True)
    return (x - m) * lax.rsqrt(v + 1e-5)


def _mm(a, b):
    return jnp.dot(a, b, preferred_element_type=jnp.float32)


def _mmb(a, b):
    return jnp.dot(a.astype(jnp.bfloat16), b.astype(jnp.bfloat16),
                   preferred_element_type=jnp.float32)


def _sc_gather(nodes, senders, receivers):
    mesh = plsc.VectorSubcoreMesh(core_axis_name="c", subcore_axis_name="s",
                                  num_cores=NCORES, num_subcores=NSUB)

    @functools.partial(
        pl.kernel,
        mesh=mesh,
        out_type=(
            jax.ShapeDtypeStruct((E, DN), jnp.float32),
            jax.ShapeDtypeStruct((E, DN), jnp.float32),
        ),
        scratch_types=[
            pltpu.VMEM((GROUP,), jnp.int32),
            pltpu.VMEM((GROUP, DN), jnp.float32),
            pltpu.SemaphoreType.DMA,
        ],
    )
    def k(nodes_hbm, sidx_hbm, ridx_hbm, sent_hbm, recv_hbm, idx_v, rows_v, sem):
        cid = lax.axis_index("c")
        sid = lax.axis_index("s")
        wid = sid * NCORES + cid
        base = NGROUPS // NW
        rem = NGROUPS % NW
        start = wid * base + jnp.minimum(wid, rem)
        cnt = base + (wid < rem).astype(jnp.int32)

        def run(idx_hbm, out_hbm):
            def body(i, carry):
                g = (start + i) * GROUP
                pltpu.sync_copy(idx_hbm.at[pl.ds(g, GROUP)], idx_v)
                pltpu.async_copy(nodes_hbm.at[idx_v], rows_v, sem).wait()
                pltpu.sync_copy(rows_v, out_hbm.at[pl.ds(g, GROUP)])
                return carry

            lax.fori_loop(0, cnt, body, 0)

        run(sidx_hbm, sent_hbm)
        run(ridx_hbm, recv_hbm)

    return k(nodes, senders, receivers)


def _sc_scatter(e_out, receivers):
    mesh = plsc.VectorSubcoreMesh(core_axis_name="c", subcore_axis_name="s",
                                  num_cores=NCORES, num_subcores=NSUB)

    @functools.partial(
        pl.kernel,
        mesh=mesh,
        out_type=jax.ShapeDtypeStruct((NW * NPAD, 128), jnp.float32),
        scratch_types=[
            pltpu.VMEM((CHUNK,), jnp.int32),
            pltpu.VMEM((CHUNK, EO), jnp.float32),
            pltpu.VMEM((PACK + 8, 128), jnp.float32),
        ],
    )
    def k(eout_hbm, ridx_hbm, out_hbm, idx_v, rows_v, acc_v):
        cid = lax.axis_index("c")
        sid = lax.axis_index("s")
        wid = sid * NCORES + cid
        ebase = wid * EPT

        def do_pass(sp, carry0):
            def zero(i, carry):
                for c8 in range(8):
                    acc_v[i, pl.ds(c8 * 16, 16)] = jnp.zeros((16,), jnp.float32)
                return carry

            lax.fori_loop(0, PACK + 8, zero, 0)

            def chunk(ch, carry):
                off = ebase + ch * CHUNK
                pltpu.sync_copy(ridx_hbm.at[pl.ds(off, CHUNK)], idx_v)
                pltpu.sync_copy(eout_hbm.at[pl.ds(off, CHUNK)], rows_v)

                def grp(kk, carry2):
                    local = idx_v[pl.ds(kk * 16, 16)] - sp * SEG
                    ebase16 = kk * 16
                    for l in range(16):
                        r = local[l]
                        ok = (r >= 0) & (r < SEG)
                        r2 = jnp.where(ok, r, SEG)
                        q = lax.shift_right_logical(r2, 3)
                        c16 = (r2 & 7) * 16
                        acc_v[q, pl.ds(c16, 16)] += rows_v[ebase16 + l, :]
                    return carry2

                lax.fori_loop(0, NGRP, grp, 0)
                return carry

            lax.fori_loop(0, NCHUNK, chunk, 0)
            pltpu.sync_copy(acc_v.at[pl.ds(0, PACK)],
                            out_hbm.at[pl.ds(wid * NPAD + sp * PACK, PACK)])
            return carry0

        lax.fori_loop(0, NPASS, do_pass, 0)

    return k(e_out, receivers).reshape(NW, NPAD, 128)


def _presum_body(p_ref, out_ref):
    out_ref[...] = jnp.sum(p_ref[...], axis=0)


def _tc_presum(packed3):
    PB = 160
    return pl.pallas_call(
        _presum_body,
        grid=(NPAD // 8 // PB,),
        in_specs=[pl.BlockSpec((NW, PB, 128), lambda i: (0, i, 0))],
        out_specs=pl.BlockSpec((PB, 128), lambda i: (i, 0)),
        out_shape=jax.ShapeDtypeStruct((NPAD // 8, 128), jnp.float32),
    )(packed3)


def _edge_body(ep_ref, edges_ref, sent_ref, recv_ref, gg_ref,
               w0_ref, b0_ref, w1_ref, b1_ref, w2_ref, b2_ref, w3_ref, b3_ref,
               eout_ref, eagg_ref):
    i = pl.program_id(0)
    ep = ep_ref[0, 0, :]
    onehot = (ep[:, None] == lax.broadcasted_iota(jnp.int32, (TE, G), 1)
              ).astype(jnp.float32)
    w0 = w0_ref[...]
    g0 = _mm(gg_ref[...], w0[272:400, :])
    h = (_mmb(edges_ref[...], w0[0:16, :])
         + _mmb(sent_ref[...], w0[16:144, :])
         + _mmb(recv_ref[...], w0[144:272, :])
         + _mm(onehot, g0)
         + b0_ref[...])
    h = jax.nn.relu(_ln(h))
    h = jax.nn.relu(_ln(_mmb(h, w1_ref[...]) + b1_ref[...]))
    h = jax.nn.relu(_ln(_mmb(h, w2_ref[...]) + b2_ref[...]))
    eo = _mmb(h, w3_ref[...]) + b3_ref[...]
    eout_ref[...] = eo

    @pl.when(i == 0)
    def _():
        eagg_ref[...] = jnp.zeros_like(eagg_ref)

    eagg_ref[...] += _mm(onehot.T, eo)


def _tc_edge(edgepos3, edges, sent, recv, gg, w0, b0, w1, b1, w2, b2, w3, b3):
    nb = E // TE
    full = lambda shape: pl.BlockSpec(shape, lambda i: (0,) * len(shape))
    return pl.pallas_call(
        _edge_body,
        grid=(nb,),
        in_specs=[
            pl.BlockSpec((1, 1, TE), lambda i: (i, 0, 0)),
            pl.BlockSpec((TE, DE), lambda i: (i, 0)),
            pl.BlockSpec((TE, DN), lambda i: (i, 0)),
            pl.BlockSpec((TE, DN), lambda i: (i, 0)),
            full((G, DN)),
            full((400, 128)), full((1, 128)),
            full((128, 128)), full((1, 128)),
            full((128, 128)), full((1, 128)),
            full((128, EO)), full((1, EO)),
        ],
        out_specs=[
            pl.BlockSpec((TE, EO), lambda i: (i, 0)),
            pl.BlockSpec((G, EO), lambda i: (0, 0)),
        ],
        out_shape=[
            jax.ShapeDtypeStruct((E, EO), jnp.float32),
            jax.ShapeDtypeStruct((G, EO), jnp.float32),
        ],
    )(edgepos3, edges, sent, recv, gg, w0, b0, w1, b1, w2, b2, w3, b3)


def _node_body(batch_ref, nodes_ref, agg_ref, gg_ref, eagg_ref,
               nw0_ref, nb0_ref, nw1_ref, nb1_ref, nw2_ref, nb2_ref,
               nw3_ref, nb3_ref,
               gw0_ref, gb0_ref, gw1_ref, gb1_ref, gw2_ref, gb2_ref,
               gw3_ref, gb3_ref,
               nout_ref, gout_ref, nacc_ref):
    i = pl.program_id(0)
    b = batch_ref[0, 0, :]
    onehot = (b[:, None] == lax.broadcasted_iota(jnp.int32, (TN, G), 1)
              ).astype(jnp.float32)
    agg = agg_ref[...]
    nw0 = nw0_ref[...]
    gn0 = _mm(gg_ref[...], nw0[144:272, :])
    h = (_mm(nodes_ref[...], nw0[0:128, :])
         + _mm(agg, nw0[128:144, :])
         + _mm(onehot, gn0)
         + nb0_ref[...])
    h = jax.nn.relu(_ln(h))
    h = jax.nn.relu(_ln(_mm(h, nw1_ref[...]) + nb1_ref[...]))
    h = jax.nn.relu(_ln(_mm(h, nw2_ref[...]) + nb2_ref[...]))
    no = _mm(h, nw3_ref[...]) + nb3_ref[...]
    nout_ref[...] = no

    @pl.when(i == 0)
    def _():
        nacc_ref[...] = jnp.zeros_like(nacc_ref)

    nacc_ref[...] += _mm(onehot.T, no)

    @pl.when(i == (N // TN) - 1)
    def _():
        gw0 = gw0_ref[...]
        gh = (_mm(nacc_ref[...], gw0[0:128, :])
              + _mm(eagg_ref[...], gw0[128:144, :])
              + _mm(gg_ref[...], gw0[144:272, :])
              + gb0_ref[...])
        gh = jax.nn.relu(_ln(gh))
        gh = jax.nn.relu(_ln(_mm(gh, gw1_ref[...]) + gb1_ref[...]))
        gh = jax.nn.relu(_ln(_mm(gh, gw2_ref[...]) + gb2_ref[...]))
        gout_ref[...] = _mm(gh, gw3_ref[...]) + gb3_ref[...]


def _tc_node(batch3, nodes, aggp, gg, eagg,
             nw0, nb0, nw1, nb1, nw2, nb2, nw3, nb3,
             gw0, gb0, gw1, gb1, gw2, gb2, gw3, gb3):
    nb = N // TN
    full = lambda shape: pl.BlockSpec(shape, lambda i: (0,) * len(shape))
    return pl.pallas_call(
        _node_body,
        grid=(nb,),
        in_specs=[
            pl.BlockSpec((1, 1, TN), lambda i: (i, 0, 0)),
            pl.BlockSpec((TN, DN), lambda i: (i, 0)),
            pl.BlockSpec((TN, EO), lambda i: (i, 0)),
            full((G, DN)),
            full((G, EO)),
            full((272, 128)), full((1, 128)),
            full((128, 128)), full((1, 128)),
            full((128, 128)), full((1, 128)),
            full((128, NO)), full((1, NO)),
            full((272, 128)), full((1, 128)),
            full((128, 128)), full((1, 128)),
            full((128, 128)), full((1, 128)),
            full((128, 128)), full((1, 128)),
        ],
        out_specs=[
            pl.BlockSpec((TN, NO), lambda i: (i, 0)),
            pl.BlockSpec((G, 128), lambda i: (0, 0)),
        ],
        out_shape=[
            jax.ShapeDtypeStruct((N, NO), jnp.float32),
            jax.ShapeDtypeStruct((G, 128), jnp.float32),
        ],
        scratch_shapes=[pltpu.VMEM((G, NO), jnp.float32)],
    )(batch3, nodes, aggp, gg, eagg,
      nw0, nb0, nw1, nb1, nw2, nb2, nw3, nb3,
      gw0, gb0, gw1, gb1, gw2, gb2, gw3, gb3)


def kernel(nodes, edges, graph_globals, senders, receivers, batch, edgepos,
           eW0, eb0, eW1, eb1, eW2, eb2, eW3, eb3,
           nW0, nb0, nW1, nb1, nW2, nb2, nW3, nb3,
           gW0, gb0, gW1, gb1, gW2, gb2, gW3, gb3):
    r2 = lambda b: b.reshape(1, -1)
    sent, recv = _sc_gather(nodes, senders, receivers)
    edgepos3 = edgepos.reshape(E // TE, 1, TE)
    e_out, eagg = _tc_edge(edgepos3, edges, sent, recv, graph_globals,
                           eW0, r2(eb0), eW1, r2(eb1), eW2, r2(eb2),
                           eW3, r2(eb3))
    packed3 = _sc_scatter(e_out, receivers)
    aggp = _tc_presum(packed3).reshape(NPAD, EO)
    batch3 = batch.reshape(N // TN, 1, TN)
    n_out, g_out = _tc_node(batch3, nodes, aggp, graph_globals, eagg,
                            nW0, r2(nb0), nW1, r2(nb1), nW2, r2(nb2),
                            nW3, r2(nb3),
                            gW0, r2(gb0), gW1, r2(gb1), gW2, r2(gb2),
                            gW3, r2(gb3))
    return (e_out, n_out, g_out)


# speedup vs baseline: 1.2845x; 1.2845x over previous
import functools

import jax
import jax.numpy as jnp
from jax import lax
from jax.experimental import pallas as pl
from jax.experimental.pallas import tpu as pltpu
from jax.experimental.pallas import tpu_sc as plsc

N = 10000
E = 320000
G = 8
DN = 128
DE = 16
EO = 16
NO = 128

GROUP = 128
NGROUPS = E // GROUP
NCORES = 2
NSUB = 16
NW = NCORES * NSUB

NPAD = 10240
NPASS = 4
SEG = NPAD // NPASS
PACK = SEG // 8
CHUNK = 400
EPT = E // NW
NCHUNK = EPT // CHUNK
NGRP = CHUNK // 16

TE = 2000
TN = 2000


def _ln(x):
    m = jnp.mean(x, axis=-1, keepdims=True)
    v = jnp.mean((x - m) ** 2, axis=-1, keepdims=True)
    return (x - m) * lax.rsqrt(v + 1e-5)


def _mm(a, b):
    return jnp.dot(a, b, preferred_element_type=jnp.float32)


def _sc_gather(nodes, senders, receivers):
    mesh = plsc.VectorSubcoreMesh(core_axis_name="c", subcore_axis_name="s",
                                  num_cores=NCORES, num_subcores=NSUB)

    @functools.partial(
        pl.kernel,
        mesh=mesh,
        out_type=(
            jax.ShapeDtypeStruct((E, DN), jnp.float32),
            jax.ShapeDtypeStruct((E, DN), jnp.float32),
        ),
        scratch_types=[
            pltpu.VMEM((2, GROUP), jnp.int32),
            pltpu.VMEM((2, GROUP, DN), jnp.float32),
            pltpu.SemaphoreType.DMA,
            pltpu.SemaphoreType.DMA,
        ],
    )
    def k(nodes_hbm, sidx_hbm, ridx_hbm, sent_hbm, recv_hbm,
          idx_v, rows_v, sem0, sem1):
        cid = lax.axis_index("c")
        sid = lax.axis_index("s")
        wid = sid * NCORES + cid
        base = NGROUPS // NW
        rem = NGROUPS % NW
        start = wid * base + jnp.minimum(wid, rem)
        cnt = base + (wid < rem).astype(jnp.int32)

        def run(idx_hbm, out_hbm):
            def fire(slot, sem, i):
                g = (start + i) * GROUP
                pltpu.sync_copy(idx_hbm.at[pl.ds(g, GROUP)], idx_v.at[slot])
                pltpu.async_copy(nodes_hbm.at[idx_v.at[slot]],
                                 rows_v.at[slot], sem)

            def drain(slot, sem, i):
                g = (start + i) * GROUP
                pltpu.make_async_copy(nodes_hbm.at[idx_v.at[slot]],
                                      rows_v.at[slot], sem).wait()
                pltpu.sync_copy(rows_v.at[slot], out_hbm.at[pl.ds(g, GROUP)])

            @pl.when(cnt > 0)
            def _():
                fire(0, sem0, 0)

            @pl.when(cnt > 1)
            def _():
                fire(1, sem1, 1)

            def pair(p, carry):
                i0 = 2 * p

                @pl.when(i0 < cnt)
                def _():
                    drain(0, sem0, i0)

                    @pl.when(i0 + 2 < cnt)
                    def _():
                        fire(0, sem0, i0 + 2)

                @pl.when(i0 + 1 < cnt)
                def _():
                    drain(1, sem1, i0 + 1)

                    @pl.when(i0 + 3 < cnt)
                    def _():
                        fire(1, sem1, i0 + 3)

                return carry

            lax.fori_loop(0, (cnt + 1) // 2, pair, 0)

        run(sidx_hbm, sent_hbm)
        run(ridx_hbm, recv_hbm)

    return k(nodes, senders, receivers)


def _sc_scatter(e_out, receivers):
    mesh = plsc.VectorSubcoreMesh(core_axis_name="c", subcore_axis_name="s",
                                  num_cores=NCORES, num_subcores=NSUB)

    @functools.partial(
        pl.kernel,
        mesh=mesh,
        out_type=jax.ShapeDtypeStruct((NW * NPAD, 128), jnp.float32),
        scratch_types=[
            pltpu.VMEM((CHUNK,), jnp.int32),
            pltpu.VMEM((CHUNK, EO), jnp.float32),
            pltpu.VMEM((PACK + 8, 128), jnp.float32),
        ],
    )
    def k(eout_hbm, ridx_hbm, out_hbm, idx_v, rows_v, acc_v):
        cid = lax.axis_index("c")
        sid = lax.axis_index("s")
        wid = sid * NCORES + cid
        ebase = wid * EPT

        def do_pass(sp, carry0):
            def zero(i, carry):
                for c8 in range(8):
                    acc_v[i, pl.ds(c8 * 16, 16)] = jnp.zeros((16,), jnp.float32)
                return carry

            lax.fori_loop(0, PACK + 8, zero, 0)

            def chunk(ch, carry):
                off = ebase + ch * CHUNK
                pltpu.sync_copy(ridx_hbm.at[pl.ds(off, CHUNK)], idx_v)
                pltpu.sync_copy(eout_hbm.at[pl.ds(off, CHUNK)], rows_v)

                def grp(kk, carry2):
                    local = idx_v[pl.ds(kk * 16, 16)] - sp * SEG
                    ebase16 = kk * 16
                    for l in range(16):
                        r = local[l]
                        ok = (r >= 0) & (r < SEG)
                        r2 = jnp.where(ok, r, SEG)
                        q = lax.shift_right_logical(r2, 3)
                        c16 = (r2 & 7) * 16
                        acc_v[q, pl.ds(c16, 16)] += rows_v[ebase16 + l, :]
                    return carry2

                lax.fori_loop(0, NGRP, grp, 0)
                return carry

            lax.fori_loop(0, NCHUNK, chunk, 0)
            pltpu.sync_copy(acc_v.at[pl.ds(0, PACK)],
                            out_hbm.at[pl.ds(wid * NPAD + sp * PACK, PACK)])
            return carry0

        lax.fori_loop(0, NPASS, do_pass, 0)

    return k(e_out, receivers).reshape(NW, NPAD, 128)


def _presum_body(p_ref, out_ref):
    out_ref[...] = jnp.sum(p_ref[...], axis=0)


def _tc_presum(packed3):
    PB = 160
    return pl.pallas_call(
        _presum_body,
        grid=(NPAD // 8 // PB,),
        in_specs=[pl.BlockSpec((NW, PB, 128), lambda i: (0, i, 0))],
        out_specs=pl.BlockSpec((PB, 128), lambda i: (i, 0)),
        out_shape=jax.ShapeDtypeStruct((NPAD // 8, 128), jnp.float32),
    )(packed3)


def _edge_body(ep_ref, edges_ref, sent_ref, recv_ref, gg_ref,
               w0_ref, b0_ref, w1_ref, b1_ref, w2_ref, b2_ref, w3_ref, b3_ref,
               eout_ref, eagg_ref):
    i = pl.program_id(0)
    ep = ep_ref[0, 0, :]
    onehot = (ep[:, None] == lax.broadcasted_iota(jnp.int32, (TE, G), 1)
              ).astype(jnp.float32)
    w0 = w0_ref[...]
    g0 = _mm(gg_ref[...], w0[272:400, :])
    h = (_mm(edges_ref[...], w0[0:16, :])
         + _mm(sent_ref[...], w0[16:144, :])
         + _mm(recv_ref[...], w0[144:272, :])
         + _mm(onehot, g0)
         + b0_ref[...])
    h = jax.nn.relu(_ln(h))
    h = jax.nn.relu(_ln(_mm(h, w1_ref[...]) + b1_ref[...]))
    h = jax.nn.relu(_ln(_mm(h, w2_ref[...]) + b2_ref[...]))
    eo = _mm(h, w3_ref[...]) + b3_ref[...]
    eout_ref[...] = eo

    @pl.when(i == 0)
    def _():
        eagg_ref[...] = jnp.zeros_like(eagg_ref)

    eagg_ref[...] += _mm(onehot.T, eo)


def _tc_edge(edgepos3, edges, sent, recv, gg, w0, b0, w1, b1, w2, b2, w3, b3):
    nb = E // TE
    full = lambda shape: pl.BlockSpec(shape, lambda i: (0,) * len(shape))
    return pl.pallas_call(
        _edge_body,
        grid=(nb,),
        in_specs=[
            pl.BlockSpec((1, 1, TE), lambda i: (i, 0, 0)),
            pl.BlockSpec((TE, DE), lambda i: (i, 0)),
            pl.BlockSpec((TE, DN), lambda i: (i, 0)),
            pl.BlockSpec((TE, DN), lambda i: (i, 0)),
            full((G, DN)),
            full((400, 128)), full((1, 128)),
            full((128, 128)), full((1, 128)),
            full((128, 128)), full((1, 128)),
            full((128, EO)), full((1, EO)),
        ],
        out_specs=[
            pl.BlockSpec((TE, EO), lambda i: (i, 0)),
            pl.BlockSpec((G, EO), lambda i: (0, 0)),
        ],
        out_shape=[
            jax.ShapeDtypeStruct((E, EO), jnp.float32),
            jax.ShapeDtypeStruct((G, EO), jnp.float32),
        ],
    )(edgepos3, edges, sent, recv, gg, w0, b0, w1, b1, w2, b2, w3, b3)


def _node_body(batch_ref, nodes_ref, agg_ref, gg_ref, eagg_ref,
               nw0_ref, nb0_ref, nw1_ref, nb1_ref, nw2_ref, nb2_ref,
               nw3_ref, nb3_ref,
               gw0_ref, gb0_ref, gw1_ref, gb1_ref, gw2_ref, gb2_ref,
               gw3_ref, gb3_ref,
               nout_ref, gout_ref, nacc_ref):
    i = pl.program_id(0)
    b = batch_ref[0, 0, :]
    onehot = (b[:, None] == lax.broadcasted_iota(jnp.int32, (TN, G), 1)
              ).astype(jnp.float32)
    agg = agg_ref[...]
    nw0 = nw0_ref[...]
    gn0 = _mm(gg_ref[...], nw0[144:272, :])
    h = (_mm(nodes_ref[...], nw0[0:128, :])
         + _mm(agg, nw0[128:144, :])
         + _mm(onehot, gn0)
         + nb0_ref[...])
    h = jax.nn.relu(_ln(h))
    h = jax.nn.relu(_ln(_mm(h, nw1_ref[...]) + nb1_ref[...]))
    h = jax.nn.relu(_ln(_mm(h, nw2_ref[...]) + nb2_ref[...]))
    no = _mm(h, nw3_ref[...]) + nb3_ref[...]
    nout_ref[...] = no

    @pl.when(i == 0)
    def _():
        nacc_ref[...] = jnp.zeros_like(nacc_ref)

    nacc_ref[...] += _mm(onehot.T, no)

    @pl.when(i == (N // TN) - 1)
    def _():
        gw0 = gw0_ref[...]
        gh = (_mm(nacc_ref[...], gw0[0:128, :])
              + _mm(eagg_ref[...], gw0[128:144, :])
              + _mm(gg_ref[...], gw0[144:272, :])
              + gb0_ref[...])
        gh = jax.nn.relu(_ln(gh))
        gh = jax.nn.relu(_ln(_mm(gh, gw1_ref[...]) + gb1_ref[...]))
        gh = jax.nn.relu(_ln(_mm(gh, gw2_ref[...]) + gb2_ref[...]))
        gout_ref[...] = _mm(gh, gw3_ref[...]) + gb3_ref[...]


def _tc_node(batch3, nodes, aggp, gg, eagg,
             nw0, nb0, nw1, nb1, nw2, nb2, nw3, nb3,
             gw0, gb0, gw1, gb1, gw2, gb2, gw3, gb3):
    nb = N // TN
    full = lambda shape: pl.BlockSpec(shape, lambda i: (0,) * len(shape))
    return pl.pallas_call(
        _node_body,
        grid=(nb,),
        in_specs=[
            pl.BlockSpec((1, 1, TN), lambda i: (i, 0, 0)),
            pl.BlockSpec((TN, DN), lambda i: (i, 0)),
            pl.BlockSpec((TN, EO), lambda i: (i, 0)),
            full((G, DN)),
            full((G, EO)),
            full((272, 128)), full((1, 128)),
            full((128, 128)), full((1, 128)),
            full((128, 128)), full((1, 128)),
            full((128, NO)), full((1, NO)),
            full((272, 128)), full((1, 128)),
            full((128, 128)), full((1, 128)),
            full((128, 128)), full((1, 128)),
            full((128, 128)), full((1, 128)),
        ],
        out_specs=[
            pl.BlockSpec((TN, NO), lambda i: (i, 0)),
            pl.BlockSpec((G, 128), lambda i: (0, 0)),
        ],
        out_shape=[
            jax.ShapeDtypeStruct((N, NO), jnp.float32),
            jax.ShapeDtypeStruct((G, 128), jnp.float32),
        ],
        scratch_shapes=[pltpu.VMEM((G, NO), jnp.float32)],
    )(batch3, nodes, aggp, gg, eagg,
      nw0, nb0, nw1, nb1, nw2, nb2, nw3, nb3,
      gw0, gb0, gw1, gb1, gw2, gb2, gw3, gb3)


def kernel(nodes, edges, graph_globals, senders, receivers, batch, edgepos,
           eW0, eb0, eW1, eb1, eW2, eb2, eW3, eb3,
           nW0, nb0, nW1, nb1, nW2, nb2, nW3, nb3,
           gW0, gb0, gW1, gb1, gW2, gb2, gW3, gb3):
    r2 = lambda b: b.reshape(1, -1)
    sent, recv = _sc_gather(nodes, senders, receivers)
    edgepos3 = edgepos.reshape(E // TE, 1, TE)
    e_out, eagg = _tc_edge(edgepos3, edges, sent, recv, graph_globals,
                           eW0, r2(eb0), eW1, r2(eb1), eW2, r2(eb2),
                           eW3, r2(eb3))
    packed3 = _sc_scatter(e_out, receivers)
    aggp = _tc_presum(packed3).reshape(NPAD, EO)
    batch3 = batch.reshape(N // TN, 1, TN)
    n_out, g_out = _tc_node(batch3, nodes, aggp, graph_globals, eagg,
                            nW0, r2(nb0), nW1, r2(nb1), nW2, r2(nb2),
                            nW3, r2(nb3),
                            gW0, r2(gb0), gW1, r2(gb1), gW2, r2(gb2),
                            gW3, r2(gb3))
    return (e_out, n_out, g_out)


# speedup vs baseline: 1.4208x; 1.1061x over previous
import functools

import jax
import jax.numpy as jnp
from jax import lax
from jax.experimental import pallas as pl
from jax.experimental.pallas import tpu as pltpu
from jax.experimental.pallas import tpu_sc as plsc

N = 10000
E = 320000
G = 8
DN = 128
DE = 16
EO = 16
NO = 128

GROUP = 128
NGROUPS = E // GROUP
NCORES = 2
NSUB = 16
NW = NCORES * NSUB

NPAD = 10240
NPASS = 3
SEG = 3456
PACK = SEG // 8
CHUNK = 400
EPT = E // NW
NCHUNK = EPT // CHUNK
NGRP = CHUNK // 16

TE = 2000
TN = 2000


def _ln(x):
    m = jnp.mean(x, axis=-1, keepdims=True)
    v = jnp.mean((x - m) ** 2, axis=-1, keepdims=True)
    return (x - m) * lax.rsqrt(v + 1e-5)


def _mm(a, b):
    return jnp.dot(a, b, preferred_element_type=jnp.float32)


def _sc_gather(nodes, senders, receivers):
    mesh = plsc.VectorSubcoreMesh(core_axis_name="c", subcore_axis_name="s",
                                  num_cores=NCORES, num_subcores=NSUB)

    @functools.partial(
        pl.kernel,
        mesh=mesh,
        out_type=(
            jax.ShapeDtypeStruct((E, DN), jnp.float32),
            jax.ShapeDtypeStruct((E, DN), jnp.float32),
        ),
        scratch_types=[
            pltpu.VMEM((2, GROUP), jnp.int32),
            pltpu.VMEM((2, GROUP, DN), jnp.float32),
            pltpu.SemaphoreType.DMA,
            pltpu.SemaphoreType.DMA,
        ],
    )
    def k(nodes_hbm, sidx_hbm, ridx_hbm, sent_hbm, recv_hbm,
          idx_v, rows_v, sem0, sem1):
        cid = lax.axis_index("c")
        sid = lax.axis_index("s")
        wid = sid * NCORES + cid
        base = NGROUPS // NW
        rem = NGROUPS % NW
        start = wid * base + jnp.minimum(wid, rem)
        cnt = base + (wid < rem).astype(jnp.int32)

        def run(idx_hbm, out_hbm):
            def fire(slot, sem, i):
                g = (start + i) * GROUP
                pltpu.sync_copy(idx_hbm.at[pl.ds(g, GROUP)], idx_v.at[slot])
                pltpu.async_copy(nodes_hbm.at[idx_v.at[slot]],
                                 rows_v.at[slot], sem)

            def drain(slot, sem, i):
                g = (start + i) * GROUP
                pltpu.make_async_copy(nodes_hbm.at[idx_v.at[slot]],
                                      rows_v.at[slot], sem).wait()
                pltpu.sync_copy(rows_v.at[slot], out_hbm.at[pl.ds(g, GROUP)])

            @pl.when(cnt > 0)
            def _():
                fire(0, sem0, 0)

            @pl.when(cnt > 1)
            def _():
                fire(1, sem1, 1)

            def pair(p, carry):
                i0 = 2 * p

                @pl.when(i0 < cnt)
                def _():
                    drain(0, sem0, i0)

                    @pl.when(i0 + 2 < cnt)
                    def _():
                        fire(0, sem0, i0 + 2)

                @pl.when(i0 + 1 < cnt)
                def _():
                    drain(1, sem1, i0 + 1)

                    @pl.when(i0 + 3 < cnt)
                    def _():
                        fire(1, sem1, i0 + 3)

                return carry

            lax.fori_loop(0, (cnt + 1) // 2, pair, 0)

        run(sidx_hbm, sent_hbm)
        run(ridx_hbm, recv_hbm)

    return k(nodes, senders, receivers)


def _sc_scatter(e_out, receivers):
    mesh = plsc.VectorSubcoreMesh(core_axis_name="c", subcore_axis_name="s",
                                  num_cores=NCORES, num_subcores=NSUB)

    @functools.partial(
        pl.kernel,
        mesh=mesh,
        out_type=jax.ShapeDtypeStruct((NW * NPAD, 128), jnp.float32),
        scratch_types=[
            pltpu.VMEM((CHUNK,), jnp.int32),
            pltpu.VMEM((CHUNK, EO), jnp.float32),
            pltpu.VMEM((PACK + 1, 128), jnp.float32),
        ],
    )
    def k(eout_hbm, ridx_hbm, out_hbm, idx_v, rows_v, acc_v):
        cid = lax.axis_index("c")
        sid = lax.axis_index("s")
        wid = sid * NCORES + cid
        ebase = wid * EPT

        def do_pass(sp, carry0):
            def zero(i, carry):
                for c8 in range(8):
                    acc_v[i, pl.ds(c8 * 16, 16)] = jnp.zeros((16,), jnp.float32)
                return carry

            lax.fori_loop(0, PACK + 1, zero, 0)

            def chunk(ch, carry):
                off = ebase + ch * CHUNK
                pltpu.sync_copy(ridx_hbm.at[pl.ds(off, CHUNK)], idx_v)
                pltpu.sync_copy(eout_hbm.at[pl.ds(off, CHUNK)], rows_v)

                def grp(kk, carry2):
                    local = idx_v[pl.ds(kk * 16, 16)] - sp * SEG
                    ebase16 = kk * 16
                    for l in range(16):
                        r = local[l]
                        ok = (r >= 0) & (r < SEG)
                        r2 = jnp.where(ok, r, SEG)
                        q = lax.shift_right_logical(r2, 3)
                        c16 = (r2 & 7) * 16
                        acc_v[q, pl.ds(c16, 16)] += rows_v[ebase16 + l, :]
                    return carry2

                lax.fori_loop(0, NGRP, grp, 0)
                return carry

            lax.fori_loop(0, NCHUNK, chunk, 0)
            pltpu.sync_copy(acc_v.at[pl.ds(0, PACK)],
                            out_hbm.at[pl.ds(wid * NPAD + sp * PACK, PACK)])
            return carry0

        lax.fori_loop(0, NPASS, do_pass, 0)

    return k(e_out, receivers).reshape(NW, NPAD, 128)


def _presum_body(p_ref, out_ref):
    out_ref[...] = jnp.sum(p_ref[...], axis=0)


def _tc_presum(packed3):
    PB = 160
    return pl.pallas_call(
        _presum_body,
        grid=(NPAD // 8 // PB,),
        in_specs=[pl.BlockSpec((NW, PB, 128), lambda i: (0, i, 0))],
        out_specs=pl.BlockSpec((PB, 128), lambda i: (i, 0)),
        out_shape=jax.ShapeDtypeStruct((NPAD // 8, 128), jnp.float32),
    )(packed3)


def _edge_body(ep_ref, edges_ref, sent_ref, recv_ref, gg_ref,
               w0_ref, b0_ref, w1_ref, b1_ref, w2_ref, b2_ref, w3_ref, b3_ref,
               eout_ref, eagg_ref):
    i = pl.program_id(0)
    ep = ep_ref[0, 0, :]
    onehot = (ep[:, None] == lax.broadcasted_iota(jnp.int32, (TE, G), 1)
              ).astype(jnp.float32)
    w0 = w0_ref[...]
    g0 = _mm(gg_ref[...], w0[272:400, :])
    h = (_mm(edges_ref[...], w0[0:16, :])
         + _mm(sent_ref[...], w0[16:144, :])
         + _mm(recv_ref[...], w0[144:272, :])
         + _mm(onehot, g0)
         + b0_ref[...])
    h = jax.nn.relu(_ln(h))
    h = jax.nn.relu(_ln(_mm(h, w1_ref[...]) + b1_ref[...]))
    h = jax.nn.relu(_ln(_mm(h, w2_ref[...]) + b2_ref[...]))
    eo = _mm(h, w3_ref[...]) + b3_ref[...]
    eout_ref[...] = eo

    @pl.when(i == 0)
    def _():
        eagg_ref[...] = jnp.zeros_like(eagg_ref)

    eagg_ref[...] += _mm(onehot.T, eo)


def _tc_edge(edgepos3, edges, sent, recv, gg, w0, b0, w1, b1, w2, b2, w3, b3):
    nb = E // TE
    full = lambda shape: pl.BlockSpec(shape, lambda i: (0,) * len(shape))
    return pl.pallas_call(
        _edge_body,
        grid=(nb,),
        in_specs=[
            pl.BlockSpec((1, 1, TE), lambda i: (i, 0, 0)),
            pl.BlockSpec((TE, DE), lambda i: (i, 0)),
            pl.BlockSpec((TE, DN), lambda i: (i, 0)),
            pl.BlockSpec((TE, DN), lambda i: (i, 0)),
            full((G, DN)),
            full((400, 128)), full((1, 128)),
            full((128, 128)), full((1, 128)),
            full((128, 128)), full((1, 128)),
            full((128, EO)), full((1, EO)),
        ],
        out_specs=[
            pl.BlockSpec((TE, EO), lambda i: (i, 0)),
            pl.BlockSpec((G, EO), lambda i: (0, 0)),
        ],
        out_shape=[
            jax.ShapeDtypeStruct((E, EO), jnp.float32),
            jax.ShapeDtypeStruct((G, EO), jnp.float32),
        ],
    )(edgepos3, edges, sent, recv, gg, w0, b0, w1, b1, w2, b2, w3, b3)


def _node_body(batch_ref, nodes_ref, agg_ref, gg_ref, eagg_ref,
               nw0_ref, nb0_ref, nw1_ref, nb1_ref, nw2_ref, nb2_ref,
               nw3_ref, nb3_ref,
               gw0_ref, gb0_ref, gw1_ref, gb1_ref, gw2_ref, gb2_ref,
               gw3_ref, gb3_ref,
               nout_ref, gout_ref, nacc_ref):
    i = pl.program_id(0)
    b = batch_ref[0, 0, :]
    onehot = (b[:, None] == lax.broadcasted_iota(jnp.int32, (TN, G), 1)
              ).astype(jnp.float32)
    agg = agg_ref[...]
    nw0 = nw0_ref[...]
    gn0 = _mm(gg_ref[...], nw0[144:272, :])
    h = (_mm(nodes_ref[...], nw0[0:128, :])
         + _mm(agg, nw0[128:144, :])
         + _mm(onehot, gn0)
         + nb0_ref[...])
    h = jax.nn.relu(_ln(h))
    h = jax.nn.relu(_ln(_mm(h, nw1_ref[...]) + nb1_ref[...]))
    h = jax.nn.relu(_ln(_mm(h, nw2_ref[...]) + nb2_ref[...]))
    no = _mm(h, nw3_ref[...]) + nb3_ref[...]
    nout_ref[...] = no

    @pl.when(i == 0)
    def _():
        nacc_ref[...] = jnp.zeros_like(nacc_ref)

    nacc_ref[...] += _mm(onehot.T, no)

    @pl.when(i == (N // TN) - 1)
    def _():
        gw0 = gw0_ref[...]
        gh = (_mm(nacc_ref[...], gw0[0:128, :])
              + _mm(eagg_ref[...], gw0[128:144, :])
              + _mm(gg_ref[...], gw0[144:272, :])
              + gb0_ref[...])
        gh = jax.nn.relu(_ln(gh))
        gh = jax.nn.relu(_ln(_mm(gh, gw1_ref[...]) + gb1_ref[...]))
        gh = jax.nn.relu(_ln(_mm(gh, gw2_ref[...]) + gb2_ref[...]))
        gout_ref[...] = _mm(gh, gw3_ref[...]) + gb3_ref[...]


def _tc_node(batch3, nodes, aggp, gg, eagg,
             nw0, nb0, nw1, nb1, nw2, nb2, nw3, nb3,
             gw0, gb0, gw1, gb1, gw2, gb2, gw3, gb3):
    nb = N // TN
    full = lambda shape: pl.BlockSpec(shape, lambda i: (0,) * len(shape))
    return pl.pallas_call(
        _node_body,
        grid=(nb,),
        in_specs=[
            pl.BlockSpec((1, 1, TN), lambda i: (i, 0, 0)),
            pl.BlockSpec((TN, DN), lambda i: (i, 0)),
            pl.BlockSpec((TN, EO), lambda i: (i, 0)),
            full((G, DN)),
            full((G, EO)),
            full((272, 128)), full((1, 128)),
            full((128, 128)), full((1, 128)),
            full((128, 128)), full((1, 128)),
            full((128, NO)), full((1, NO)),
            full((272, 128)), full((1, 128)),
            full((128, 128)), full((1, 128)),
            full((128, 128)), full((1, 128)),
            full((128, 128)), full((1, 128)),
        ],
        out_specs=[
            pl.BlockSpec((TN, NO), lambda i: (i, 0)),
            pl.BlockSpec((G, 128), lambda i: (0, 0)),
        ],
        out_shape=[
            jax.ShapeDtypeStruct((N, NO), jnp.float32),
            jax.ShapeDtypeStruct((G, 128), jnp.float32),
        ],
        scratch_shapes=[pltpu.VMEM((G, NO), jnp.float32)],
    )(batch3, nodes, aggp, gg, eagg,
      nw0, nb0, nw1, nb1, nw2, nb2, nw3, nb3,
      gw0, gb0, gw1, gb1, gw2, gb2, gw3, gb3)


def kernel(nodes, edges, graph_globals, senders, receivers, batch, edgepos,
           eW0, eb0, eW1, eb1, eW2, eb2, eW3, eb3,
           nW0, nb0, nW1, nb1, nW2, nb2, nW3, nb3,
           gW0, gb0, gW1, gb1, gW2, gb2, gW3, gb3):
    r2 = lambda b: b.reshape(1, -1)
    sent, recv = _sc_gather(nodes, senders, receivers)
    edgepos3 = edgepos.reshape(E // TE, 1, TE)
    e_out, eagg = _tc_edge(edgepos3, edges, sent, recv, graph_globals,
                           eW0, r2(eb0), eW1, r2(eb1), eW2, r2(eb2),
                           eW3, r2(eb3))
    packed3 = _sc_scatter(e_out, receivers)
    aggp = _tc_presum(packed3).reshape(NPAD, EO)
    batch3 = batch.reshape(N // TN, 1, TN)
    n_out, g_out = _tc_node(batch3, nodes, aggp, graph_globals, eagg,
                            nW0, r2(nb0), nW1, r2(nb1), nW2, r2(nb2),
                            nW3, r2(nb3),
                            gW0, r2(gb0), gW1, r2(gb1), gW2, r2(gb2),
                            gW3, r2(gb3))
    return (e_out, n_out, g_out)


# speedup vs baseline: 1.4763x; 1.0391x over previous
import functools

import jax
import jax.numpy as jnp
from jax import lax
from jax.experimental import pallas as pl
from jax.experimental.pallas import tpu as pltpu
from jax.experimental.pallas import tpu_sc as plsc

N = 10000
E = 320000
G = 8
DN = 128
DE = 16
EO = 16
NO = 128

GROUP = 128
NGROUPS = E // GROUP
NCORES = 2
NSUB = 16
NW = NCORES * NSUB

NPAD = 10240
NPASS = 3
SEG = 3456
PACK = SEG // 8
CHUNK = 400
EPT = E // NW
NCHUNK = EPT // CHUNK
NGRP = CHUNK // 16

TE = 4000
TN = 2000


def _ln(x):
    m = jnp.mean(x, axis=-1, keepdims=True)
    v = jnp.mean((x - m) ** 2, axis=-1, keepdims=True)
    return (x - m) * lax.rsqrt(v + 1e-5)


def _mm(a, b):
    return jnp.dot(a, b, preferred_element_type=jnp.float32)


def _sc_gather(nodes, senders, receivers):
    mesh = plsc.VectorSubcoreMesh(core_axis_name="c", subcore_axis_name="s",
                                  num_cores=NCORES, num_subcores=NSUB)

    @functools.partial(
        pl.kernel,
        mesh=mesh,
        out_type=(
            jax.ShapeDtypeStruct((E, DN), jnp.float32),
            jax.ShapeDtypeStruct((E, DN), jnp.float32),
        ),
        scratch_types=[
            pltpu.VMEM((2, GROUP), jnp.int32),
            pltpu.VMEM((2, GROUP, DN), jnp.float32),
            pltpu.SemaphoreType.DMA,
            pltpu.SemaphoreType.DMA,
        ],
    )
    def k(nodes_hbm, sidx_hbm, ridx_hbm, sent_hbm, recv_hbm,
          idx_v, rows_v, sem0, sem1):
        cid = lax.axis_index("c")
        sid = lax.axis_index("s")
        wid = sid * NCORES + cid
        base = NGROUPS // NW
        rem = NGROUPS % NW
        start = wid * base + jnp.minimum(wid, rem)
        cnt = base + (wid < rem).astype(jnp.int32)

        def run(idx_hbm, out_hbm):
            def fire(slot, sem, i):
                g = (start + i) * GROUP
                pltpu.sync_copy(idx_hbm.at[pl.ds(g, GROUP)], idx_v.at[slot])
                pltpu.async_copy(nodes_hbm.at[idx_v.at[slot]],
                                 rows_v.at[slot], sem)

            def drain(slot, sem, i):
                g = (start + i) * GROUP
                pltpu.make_async_copy(nodes_hbm.at[idx_v.at[slot]],
                                      rows_v.at[slot], sem).wait()
                pltpu.sync_copy(rows_v.at[slot], out_hbm.at[pl.ds(g, GROUP)])

            @pl.when(cnt > 0)
            def _():
                fire(0, sem0, 0)

            @pl.when(cnt > 1)
            def _():
                fire(1, sem1, 1)

            def pair(p, carry):
                i0 = 2 * p

                @pl.when(i0 < cnt)
                def _():
                    drain(0, sem0, i0)

                    @pl.when(i0 + 2 < cnt)
                    def _():
                        fire(0, sem0, i0 + 2)

                @pl.when(i0 + 1 < cnt)
                def _():
                    drain(1, sem1, i0 + 1)

                    @pl.when(i0 + 3 < cnt)
                    def _():
                        fire(1, sem1, i0 + 3)

                return carry

            lax.fori_loop(0, (cnt + 1) // 2, pair, 0)

        run(sidx_hbm, sent_hbm)
        run(ridx_hbm, recv_hbm)

    return k(nodes, senders, receivers)


def _sc_scatter(e_out, receivers):
    mesh = plsc.VectorSubcoreMesh(core_axis_name="c", subcore_axis_name="s",
                                  num_cores=NCORES, num_subcores=NSUB)

    @functools.partial(
        pl.kernel,
        mesh=mesh,
        out_type=jax.ShapeDtypeStruct((NW * NPAD, 128), jnp.float32),
        scratch_types=[
            pltpu.VMEM((CHUNK,), jnp.int32),
            pltpu.VMEM((CHUNK, EO), jnp.float32),
            pltpu.VMEM((PACK + 1, 128), jnp.float32),
        ],
    )
    def k(eout_hbm, ridx_hbm, out_hbm, idx_v, rows_v, acc_v):
        cid = lax.axis_index("c")
        sid = lax.axis_index("s")
        wid = sid * NCORES + cid
        ebase = wid * EPT

        def do_pass(sp, carry0):
            def zero(i, carry):
                for c8 in range(8):
                    acc_v[i, pl.ds(c8 * 16, 16)] = jnp.zeros((16,), jnp.float32)
                return carry

            lax.fori_loop(0, PACK + 1, zero, 0)

            def chunk(ch, carry):
                off = ebase + ch * CHUNK
                pltpu.sync_copy(ridx_hbm.at[pl.ds(off, CHUNK)], idx_v)
                pltpu.sync_copy(eout_hbm.at[pl.ds(off, CHUNK)], rows_v)

                def grp(kk, carry2):
                    local = idx_v[pl.ds(kk * 16, 16)] - sp * SEG
                    ebase16 = kk * 16
                    for l in range(16):
                        r = local[l]
                        ok = (r >= 0) & (r < SEG)
                        r2 = jnp.where(ok, r, SEG)
                        q = lax.shift_right_logical(r2, 3)
                        c16 = (r2 & 7) * 16
                        acc_v[q, pl.ds(c16, 16)] += rows_v[ebase16 + l, :]
                    return carry2

                lax.fori_loop(0, NGRP, grp, 0)
                return carry

            lax.fori_loop(0, NCHUNK, chunk, 0)
            pltpu.sync_copy(acc_v.at[pl.ds(0, PACK)],
                            out_hbm.at[pl.ds(wid * NPAD + sp * PACK, PACK)])
            return carry0

        lax.fori_loop(0, NPASS, do_pass, 0)

    return k(e_out, receivers).reshape(NW, NPAD, 128)


def _presum_body(p_ref, out_ref):
    out_ref[...] = jnp.sum(p_ref[...], axis=0)


def _tc_presum(packed3):
    PB = 160
    return pl.pallas_call(
        _presum_body,
        grid=(NPAD // 8 // PB,),
        in_specs=[pl.BlockSpec((NW, PB, 128), lambda i: (0, i, 0))],
        out_specs=pl.BlockSpec((PB, 128), lambda i: (i, 0)),
        out_shape=jax.ShapeDtypeStruct((NPAD // 8, 128), jnp.float32),
    )(packed3)


def _edge_body(ep_ref, edges_ref, sent_ref, recv_ref, gg_ref,
               w0_ref, b0_ref, w1_ref, b1_ref, w2_ref, b2_ref, w3_ref, b3_ref,
               eout_ref, eagg_ref):
    i = pl.program_id(0)
    ep = ep_ref[0, 0, :]
    onehot = (ep[:, None] == lax.broadcasted_iota(jnp.int32, (TE, G), 1)
              ).astype(jnp.float32)
    w0 = w0_ref[...]
    g0 = _mm(gg_ref[...], w0[272:400, :])
    h = (_mm(edges_ref[...], w0[0:16, :])
         + _mm(sent_ref[...], w0[16:144, :])
         + _mm(recv_ref[...], w0[144:272, :])
         + _mm(onehot, g0)
         + b0_ref[...])
    h = jax.nn.relu(_ln(h))
    h = jax.nn.relu(_ln(_mm(h, w1_ref[...]) + b1_ref[...]))
    h = jax.nn.relu(_ln(_mm(h, w2_ref[...]) + b2_ref[...]))
    eo = _mm(h, w3_ref[...]) + b3_ref[...]
    eout_ref[...] = eo

    @pl.when(i == 0)
    def _():
        eagg_ref[...] = jnp.zeros_like(eagg_ref)

    eagg_ref[...] += _mm(onehot.T, eo)


def _tc_edge(edgepos3, edges, sent, recv, gg, w0, b0, w1, b1, w2, b2, w3, b3):
    nb = E // TE
    full = lambda shape: pl.BlockSpec(shape, lambda i: (0,) * len(shape))
    return pl.pallas_call(
        _edge_body,
        grid=(nb,),
        in_specs=[
            pl.BlockSpec((1, 1, TE), lambda i: (i, 0, 0)),
            pl.BlockSpec((TE, DE), lambda i: (i, 0)),
            pl.BlockSpec((TE, DN), lambda i: (i, 0)),
            pl.BlockSpec((TE, DN), lambda i: (i, 0)),
            full((G, DN)),
            full((400, 128)), full((1, 128)),
            full((128, 128)), full((1, 128)),
            full((128, 128)), full((1, 128)),
            full((128, EO)), full((1, EO)),
        ],
        out_specs=[
            pl.BlockSpec((TE, EO), lambda i: (i, 0)),
            pl.BlockSpec((G, EO), lambda i: (0, 0)),
        ],
        out_shape=[
            jax.ShapeDtypeStruct((E, EO), jnp.float32),
            jax.ShapeDtypeStruct((G, EO), jnp.float32),
        ],
    )(edgepos3, edges, sent, recv, gg, w0, b0, w1, b1, w2, b2, w3, b3)


def _node_body(batch_ref, nodes_ref, agg_ref, gg_ref, eagg_ref,
               nw0_ref, nb0_ref, nw1_ref, nb1_ref, nw2_ref, nb2_ref,
               nw3_ref, nb3_ref,
               gw0_ref, gb0_ref, gw1_ref, gb1_ref, gw2_ref, gb2_ref,
               gw3_ref, gb3_ref,
               nout_ref, gout_ref, nacc_ref):
    i = pl.program_id(0)
    b = batch_ref[0, 0, :]
    onehot = (b[:, None] == lax.broadcasted_iota(jnp.int32, (TN, G), 1)
              ).astype(jnp.float32)
    agg = agg_ref[...]
    nw0 = nw0_ref[...]
    gn0 = _mm(gg_ref[...], nw0[144:272, :])
    h = (_mm(nodes_ref[...], nw0[0:128, :])
         + _mm(agg, nw0[128:144, :])
         + _mm(onehot, gn0)
         + nb0_ref[...])
    h = jax.nn.relu(_ln(h))
    h = jax.nn.relu(_ln(_mm(h, nw1_ref[...]) + nb1_ref[...]))
    h = jax.nn.relu(_ln(_mm(h, nw2_ref[...]) + nb2_ref[...]))
    no = _mm(h, nw3_ref[...]) + nb3_ref[...]
    nout_ref[...] = no

    @pl.when(i == 0)
    def _():
        nacc_ref[...] = jnp.zeros_like(nacc_ref)

    nacc_ref[...] += _mm(onehot.T, no)

    @pl.when(i == (N // TN) - 1)
    def _():
        gw0 = gw0_ref[...]
        gh = (_mm(nacc_ref[...], gw0[0:128, :])
              + _mm(eagg_ref[...], gw0[128:144, :])
              + _mm(gg_ref[...], gw0[144:272, :])
              + gb0_ref[...])
        gh = jax.nn.relu(_ln(gh))
        gh = jax.nn.relu(_ln(_mm(gh, gw1_ref[...]) + gb1_ref[...]))
        gh = jax.nn.relu(_ln(_mm(gh, gw2_ref[...]) + gb2_ref[...]))
        gout_ref[...] = _mm(gh, gw3_ref[...]) + gb3_ref[...]


def _tc_node(batch3, nodes, aggp, gg, eagg,
             nw0, nb0, nw1, nb1, nw2, nb2, nw3, nb3,
             gw0, gb0, gw1, gb1, gw2, gb2, gw3, gb3):
    nb = N // TN
    full = lambda shape: pl.BlockSpec(shape, lambda i: (0,) * len(shape))
    return pl.pallas_call(
        _node_body,
        grid=(nb,),
        in_specs=[
            pl.BlockSpec((1, 1, TN), lambda i: (i, 0, 0)),
            pl.BlockSpec((TN, DN), lambda i: (i, 0)),
            pl.BlockSpec((TN, EO), lambda i: (i, 0)),
            full((G, DN)),
            full((G, EO)),
            full((272, 128)), full((1, 128)),
            full((128, 128)), full((1, 128)),
            full((128, 128)), full((1, 128)),
            full((128, NO)), full((1, NO)),
            full((272, 128)), full((1, 128)),
            full((128, 128)), full((1, 128)),
            full((128, 128)), full((1, 128)),
            full((128, 128)), full((1, 128)),
        ],
        out_specs=[
            pl.BlockSpec((TN, NO), lambda i: (i, 0)),
            pl.BlockSpec((G, 128), lambda i: (0, 0)),
        ],
        out_shape=[
            jax.ShapeDtypeStruct((N, NO), jnp.float32),
            jax.ShapeDtypeStruct((G, 128), jnp.float32),
        ],
        scratch_shapes=[pltpu.VMEM((G, NO), jnp.float32)],
    )(batch3, nodes, aggp, gg, eagg,
      nw0, nb0, nw1, nb1, nw2, nb2, nw3, nb3,
      gw0, gb0, gw1, gb1, gw2, gb2, gw3, gb3)


def kernel(nodes, edges, graph_globals, senders, receivers, batch, edgepos,
           eW0, eb0, eW1, eb1, eW2, eb2, eW3, eb3,
           nW0, nb0, nW1, nb1, nW2, nb2, nW3, nb3,
           gW0, gb0, gW1, gb1, gW2, gb2, gW3, gb3):
    r2 = lambda b: b.reshape(1, -1)
    sent, recv = _sc_gather(nodes, senders, receivers)
    edgepos3 = edgepos.reshape(E // TE, 1, TE)
    e_out, eagg = _tc_edge(edgepos3, edges, sent, recv, graph_globals,
                           eW0, r2(eb0), eW1, r2(eb1), eW2, r2(eb2),
                           eW3, r2(eb3))
    packed3 = _sc_scatter(e_out, receivers)
    aggp = _tc_presum(packed3).reshape(NPAD, EO)
    batch3 = batch.reshape(N // TN, 1, TN)
    n_out, g_out = _tc_node(batch3, nodes, aggp, graph_globals, eagg,
                            nW0, r2(nb0), nW1, r2(nb1), nW2, r2(nb2),
                            nW3, r2(nb3),
                            gW0, r2(gb0), gW1, r2(gb1), gW2, r2(gb2),
                            gW3, r2(gb3))
    return (e_out, n_out, g_out)


# speedup vs baseline: 1.4999x; 1.0160x over previous
import functools

import jax
import jax.numpy as jnp
from jax import lax
from jax.experimental import pallas as pl
from jax.experimental.pallas import tpu as pltpu
from jax.experimental.pallas import tpu_sc as plsc

N = 10000
E = 320000
G = 8
DN = 128
DE = 16
EO = 16
NO = 128

GROUP = 128
NGROUPS = E // GROUP
NCORES = 2
NSUB = 16
NW = NCORES * NSUB

NPAD = 10240
NPASS = 3
SEG = 3456
PACK = SEG // 8
CHUNK = 400
EPT = E // NW
NCHUNK = EPT // CHUNK
NGRP = CHUNK // 16

TE = 8000
TN = 2000


def _ln(x):
    m = jnp.mean(x, axis=-1, keepdims=True)
    v = jnp.mean((x - m) ** 2, axis=-1, keepdims=True)
    return (x - m) * lax.rsqrt(v + 1e-5)


def _mm(a, b):
    return jnp.dot(a, b, preferred_element_type=jnp.float32)


def _sc_gather(nodes, senders, receivers):
    mesh = plsc.VectorSubcoreMesh(core_axis_name="c", subcore_axis_name="s",
                                  num_cores=NCORES, num_subcores=NSUB)

    @functools.partial(
        pl.kernel,
        mesh=mesh,
        out_type=(
            jax.ShapeDtypeStruct((E, DN), jnp.float32),
            jax.ShapeDtypeStruct((E, DN), jnp.float32),
        ),
        scratch_types=[
            pltpu.VMEM((2, GROUP), jnp.int32),
            pltpu.VMEM((2, GROUP, DN), jnp.float32),
            pltpu.SemaphoreType.DMA,
            pltpu.SemaphoreType.DMA,
        ],
    )
    def k(nodes_hbm, sidx_hbm, ridx_hbm, sent_hbm, recv_hbm,
          idx_v, rows_v, sem0, sem1):
        cid = lax.axis_index("c")
        sid = lax.axis_index("s")
        wid = sid * NCORES + cid
        base = NGROUPS // NW
        rem = NGROUPS % NW
        start = wid * base + jnp.minimum(wid, rem)
        cnt = base + (wid < rem).astype(jnp.int32)

        def run(idx_hbm, out_hbm):
            def fire(slot, sem, i):
                g = (start + i) * GROUP
                pltpu.sync_copy(idx_hbm.at[pl.ds(g, GROUP)], idx_v.at[slot])
                pltpu.async_copy(nodes_hbm.at[idx_v.at[slot]],
                                 rows_v.at[slot], sem)

            def drain(slot, sem, i):
                g = (start + i) * GROUP
                pltpu.make_async_copy(nodes_hbm.at[idx_v.at[slot]],
                                      rows_v.at[slot], sem).wait()
                pltpu.sync_copy(rows_v.at[slot], out_hbm.at[pl.ds(g, GROUP)])

            @pl.when(cnt > 0)
            def _():
                fire(0, sem0, 0)

            @pl.when(cnt > 1)
            def _():
                fire(1, sem1, 1)

            def pair(p, carry):
                i0 = 2 * p

                @pl.when(i0 < cnt)
                def _():
                    drain(0, sem0, i0)

                    @pl.when(i0 + 2 < cnt)
                    def _():
                        fire(0, sem0, i0 + 2)

                @pl.when(i0 + 1 < cnt)
                def _():
                    drain(1, sem1, i0 + 1)

                    @pl.when(i0 + 3 < cnt)
                    def _():
                        fire(1, sem1, i0 + 3)

                return carry

            lax.fori_loop(0, (cnt + 1) // 2, pair, 0)

        run(sidx_hbm, sent_hbm)
        run(ridx_hbm, recv_hbm)

    return k(nodes, senders, receivers)


def _sc_scatter(e_out, receivers):
    mesh = plsc.VectorSubcoreMesh(core_axis_name="c", subcore_axis_name="s",
                                  num_cores=NCORES, num_subcores=NSUB)

    @functools.partial(
        pl.kernel,
        mesh=mesh,
        out_type=jax.ShapeDtypeStruct((NW * NPAD, 128), jnp.float32),
        scratch_types=[
            pltpu.VMEM((CHUNK,), jnp.int32),
            pltpu.VMEM((CHUNK, EO), jnp.float32),
            pltpu.VMEM((PACK + 1, 128), jnp.float32),
        ],
    )
    def k(eout_hbm, ridx_hbm, out_hbm, idx_v, rows_v, acc_v):
        cid = lax.axis_index("c")
        sid = lax.axis_index("s")
        wid = sid * NCORES + cid
        ebase = wid * EPT

        def do_pass(sp, carry0):
            def zero(i, carry):
                for c8 in range(8):
                    acc_v[i, pl.ds(c8 * 16, 16)] = jnp.zeros((16,), jnp.float32)
                return carry

            lax.fori_loop(0, PACK + 1, zero, 0)

            def chunk(ch, carry):
                off = ebase + ch * CHUNK
                pltpu.sync_copy(ridx_hbm.at[pl.ds(off, CHUNK)], idx_v)
                pltpu.sync_copy(eout_hbm.at[pl.ds(off, CHUNK)], rows_v)

                def grp(kk, carry2):
                    local = idx_v[pl.ds(kk * 16, 16)] - sp * SEG
                    ebase16 = kk * 16
                    for l in range(16):
                        r = local[l]
                        ok = (r >= 0) & (r < SEG)
                        r2 = jnp.where(ok, r, SEG)
                        q = lax.shift_right_logical(r2, 3)
                        c16 = (r2 & 7) * 16
                        acc_v[q, pl.ds(c16, 16)] += rows_v[ebase16 + l, :]
                    return carry2

                lax.fori_loop(0, NGRP, grp, 0)
                return carry

            lax.fori_loop(0, NCHUNK, chunk, 0)
            pltpu.sync_copy(acc_v.at[pl.ds(0, PACK)],
                            out_hbm.at[pl.ds(wid * NPAD + sp * PACK, PACK)])
            return carry0

        lax.fori_loop(0, NPASS, do_pass, 0)

    return k(e_out, receivers).reshape(NW, NPAD, 128)


def _presum_body(p_ref, out_ref):
    out_ref[...] = jnp.sum(p_ref[...], axis=0)


def _tc_presum(packed3):
    PB = 160
    return pl.pallas_call(
        _presum_body,
        grid=(NPAD // 8 // PB,),
        in_specs=[pl.BlockSpec((NW, PB, 128), lambda i: (0, i, 0))],
        out_specs=pl.BlockSpec((PB, 128), lambda i: (i, 0)),
        out_shape=jax.ShapeDtypeStruct((NPAD // 8, 128), jnp.float32),
    )(packed3)


def _edge_body(ep_ref, edges_ref, sent_ref, recv_ref, gg_ref,
               w0_ref, b0_ref, w1_ref, b1_ref, w2_ref, b2_ref, w3_ref, b3_ref,
               eout_ref, eagg_ref):
    i = pl.program_id(0)
    ep = ep_ref[0, 0, :]
    onehot = (ep[:, None] == lax.broadcasted_iota(jnp.int32, (TE, G), 1)
              ).astype(jnp.float32)
    w0 = w0_ref[...]
    g0 = _mm(gg_ref[...], w0[272:400, :])
    h = (_mm(edges_ref[...], w0[0:16, :])
         + _mm(sent_ref[...], w0[16:144, :])
         + _mm(recv_ref[...], w0[144:272, :])
         + _mm(onehot, g0)
         + b0_ref[...])
    h = jax.nn.relu(_ln(h))
    h = jax.nn.relu(_ln(_mm(h, w1_ref[...]) + b1_ref[...]))
    h = jax.nn.relu(_ln(_mm(h, w2_ref[...]) + b2_ref[...]))
    eo = _mm(h, w3_ref[...]) + b3_ref[...]
    eout_ref[...] = eo

    @pl.when(i == 0)
    def _():
        eagg_ref[...] = jnp.zeros_like(eagg_ref)

    eagg_ref[...] += _mm(onehot.T, eo)


def _tc_edge(edgepos3, edges, sent, recv, gg, w0, b0, w1, b1, w2, b2, w3, b3):
    nb = E // TE
    full = lambda shape: pl.BlockSpec(shape, lambda i: (0,) * len(shape))
    return pl.pallas_call(
        _edge_body,
        grid=(nb,),
        in_specs=[
            pl.BlockSpec((1, 1, TE), lambda i: (i, 0, 0)),
            pl.BlockSpec((TE, DE), lambda i: (i, 0)),
            pl.BlockSpec((TE, DN), lambda i: (i, 0)),
            pl.BlockSpec((TE, DN), lambda i: (i, 0)),
            full((G, DN)),
            full((400, 128)), full((1, 128)),
            full((128, 128)), full((1, 128)),
            full((128, 128)), full((1, 128)),
            full((128, EO)), full((1, EO)),
        ],
        out_specs=[
            pl.BlockSpec((TE, EO), lambda i: (i, 0)),
            pl.BlockSpec((G, EO), lambda i: (0, 0)),
        ],
        out_shape=[
            jax.ShapeDtypeStruct((E, EO), jnp.float32),
            jax.ShapeDtypeStruct((G, EO), jnp.float32),
        ],
    )(edgepos3, edges, sent, recv, gg, w0, b0, w1, b1, w2, b2, w3, b3)


def _node_body(batch_ref, nodes_ref, agg_ref, gg_ref, eagg_ref,
               nw0_ref, nb0_ref, nw1_ref, nb1_ref, nw2_ref, nb2_ref,
               nw3_ref, nb3_ref,
               gw0_ref, gb0_ref, gw1_ref, gb1_ref, gw2_ref, gb2_ref,
               gw3_ref, gb3_ref,
               nout_ref, gout_ref, nacc_ref):
    i = pl.program_id(0)
    b = batch_ref[0, 0, :]
    onehot = (b[:, None] == lax.broadcasted_iota(jnp.int32, (TN, G), 1)
              ).astype(jnp.float32)
    agg = agg_ref[...]
    nw0 = nw0_ref[...]
    gn0 = _mm(gg_ref[...], nw0[144:272, :])
    h = (_mm(nodes_ref[...], nw0[0:128, :])
         + _mm(agg, nw0[128:144, :])
         + _mm(onehot, gn0)
         + nb0_ref[...])
    h = jax.nn.relu(_ln(h))
    h = jax.nn.relu(_ln(_mm(h, nw1_ref[...]) + nb1_ref[...]))
    h = jax.nn.relu(_ln(_mm(h, nw2_ref[...]) + nb2_ref[...]))
    no = _mm(h, nw3_ref[...]) + nb3_ref[...]
    nout_ref[...] = no

    @pl.when(i == 0)
    def _():
        nacc_ref[...] = jnp.zeros_like(nacc_ref)

    nacc_ref[...] += _mm(onehot.T, no)

    @pl.when(i == (N // TN) - 1)
    def _():
        gw0 = gw0_ref[...]
        gh = (_mm(nacc_ref[...], gw0[0:128, :])
              + _mm(eagg_ref[...], gw0[128:144, :])
              + _mm(gg_ref[...], gw0[144:272, :])
              + gb0_ref[...])
        gh = jax.nn.relu(_ln(gh))
        gh = jax.nn.relu(_ln(_mm(gh, gw1_ref[...]) + gb1_ref[...]))
        gh = jax.nn.relu(_ln(_mm(gh, gw2_ref[...]) + gb2_ref[...]))
        gout_ref[...] = _mm(gh, gw3_ref[...]) + gb3_ref[...]


def _tc_node(batch3, nodes, aggp, gg, eagg,
             nw0, nb0, nw1, nb1, nw2, nb2, nw3, nb3,
             gw0, gb0, gw1, gb1, gw2, gb2, gw3, gb3):
    nb = N // TN
    full = lambda shape: pl.BlockSpec(shape, lambda i: (0,) * len(shape))
    return pl.pallas_call(
        _node_body,
        grid=(nb,),
        in_specs=[
            pl.BlockSpec((1, 1, TN), lambda i: (i, 0, 0)),
            pl.BlockSpec((TN, DN), lambda i: (i, 0)),
            pl.BlockSpec((TN, EO), lambda i: (i, 0)),
            full((G, DN)),
            full((G, EO)),
            full((272, 128)), full((1, 128)),
            full((128, 128)), full((1, 128)),
            full((128, 128)), full((1, 128)),
            full((128, NO)), full((1, NO)),
            full((272, 128)), full((1, 128)),
            full((128, 128)), full((1, 128)),
            full((128, 128)), full((1, 128)),
            full((128, 128)), full((1, 128)),
        ],
        out_specs=[
            pl.BlockSpec((TN, NO), lambda i: (i, 0)),
            pl.BlockSpec((G, 128), lambda i: (0, 0)),
        ],
        out_shape=[
            jax.ShapeDtypeStruct((N, NO), jnp.float32),
            jax.ShapeDtypeStruct((G, 128), jnp.float32),
        ],
        scratch_shapes=[pltpu.VMEM((G, NO), jnp.float32)],
    )(batch3, nodes, aggp, gg, eagg,
      nw0, nb0, nw1, nb1, nw2, nb2, nw3, nb3,
      gw0, gb0, gw1, gb1, gw2, gb2, gw3, gb3)


def kernel(nodes, edges, graph_globals, senders, receivers, batch, edgepos,
           eW0, eb0, eW1, eb1, eW2, eb2, eW3, eb3,
           nW0, nb0, nW1, nb1, nW2, nb2, nW3, nb3,
           gW0, gb0, gW1, gb1, gW2, gb2, gW3, gb3):
    r2 = lambda b: b.reshape(1, -1)
    sent, recv = _sc_gather(nodes, senders, receivers)
    edgepos3 = edgepos.reshape(E // TE, 1, TE)
    e_out, eagg = _tc_edge(edgepos3, edges, sent, recv, graph_globals,
                           eW0, r2(eb0), eW1, r2(eb1), eW2, r2(eb2),
                           eW3, r2(eb3))
    packed3 = _sc_scatter(e_out, receivers)
    aggp = _tc_presum(packed3).reshape(NPAD, EO)
    batch3 = batch.reshape(N // TN, 1, TN)
    n_out, g_out = _tc_node(batch3, nodes, aggp, graph_globals, eagg,
                            nW0, r2(nb0), nW1, r2(nb1), nW2, r2(nb2),
                            nW3, r2(nb3),
                            gW0, r2(gb0), gW1, r2(gb1), gW2, r2(gb2),
                            gW3, r2(gb3))
    return (e_out, n_out, g_out)
